# Initial kernel scaffold; baseline (speedup 1.0000x reference)
import jax
import jax.numpy as jnp
from jax.experimental import pallas as pl


def kernel(x):
    raise NotImplementedError("write your pallas kernel here")


# speedup vs baseline: 3.8684x; 3.8684x over previous
import jax
import jax.numpy as jnp
from jax.experimental import pallas as pl
from jax.experimental.pallas import tpu as pltpu

_BLOCK = 512


def _body(x_ref, o_ref, carry_ref):
    j = pl.program_id(0)

    @pl.when(j == 0)
    def _init():
        carry_ref[...] = jnp.zeros_like(carry_ref)

    x = x_ref[...]
    b = x.shape[1]
    rows = jax.lax.broadcasted_iota(jnp.int32, (b, b), 0)
    cols = jax.lax.broadcasted_iota(jnp.int32, (b, b), 1)
    tri = (rows <= cols).astype(jnp.float32)
    s = jnp.dot(x, tri, preferred_element_type=jnp.float32)
    out = s + carry_ref[:, :1]
    o_ref[...] = out
    carry_ref[:, :1] = out[:, -1:]


def kernel(x):
    m, n = x.shape
    grid = (n // _BLOCK,)
    return pl.pallas_call(
        _body,
        grid=grid,
        in_specs=[pl.BlockSpec((m, _BLOCK), lambda j: (0, j))],
        out_specs=pl.BlockSpec((m, _BLOCK), lambda j: (0, j)),
        out_shape=jax.ShapeDtypeStruct((m, n), jnp.float32),
        scratch_shapes=[pltpu.VMEM((m, 1), jnp.float32)],
    )(x)
